# speedup vs baseline: 1.0357x; 1.0357x over previous
import jax
import jax.numpy as jnp
from jax.experimental import pallas as pl

N = 10000
E = 160000
D_IN = 128
D = 256
H = 8
C = D // H
L = 2


def _layer_norm(x, g, b):
    mu = x.mean(-1, keepdims=True)
    var = ((x - mu) ** 2).mean(-1, keepdims=True)
    return g * (x - mu) / jnp.sqrt(var + 1e-5) + b


def _gatv2_conv(x, src, dst, Wl_, bl_, Wr_, br_, att_, bias_):
    n = x.shape[0]
    xl = (x @ Wl_ + bl_).reshape(n, H, C)
    xr = (x @ Wr_ + br_).reshape(n, H, C)
    e = jax.nn.leaky_relu(xl[src] + xr[dst], 0.2)
    score = (e * att_[None]).sum(-1)
    ex = jnp.exp(score)
    denom = jax.ops.segment_sum(ex, dst, num_segments=n)
    alpha = ex / (denom[dst] + 1e-16)
    out = jax.ops.segment_sum(alpha[:, :, None] * xl[src], dst, num_segments=n)
    asum = denom / (denom + 1e-16)
    return out.reshape(n, H * C) + bias_, asum.mean(-1)


def _epilogue_kernel(h_ref, wg_ref, bg_ref, ns_ref, gate_ref, gh_ref, ns_out_ref):
    h = h_ref[...]
    gate = jax.nn.sigmoid(h @ wg_ref[...] + bg_ref[0, 0])[:, 0]
    gate_ref[...] = gate[:, None]
    gh_ref[...] = h * gate[:, None]
    ns = ns_ref[...]
    mx = jnp.max(ns)
    ns_out_ref[...] = jnp.where(mx > 0, ns / mx, ns)


def kernel(x, query_embedding, W_in, b_in, W_q, b_q, Wl, bl, Wr, br, att,
           bias_conv, ln_gamma, ln_beta, W_gate, b_gate,
           edge_index_spatial, edge_index_temporal):
    n = x.shape[0]
    q = query_embedding @ W_q + b_q
    h = x @ W_in + b_in + q[None, :]
    edge_lists = [(edge_index_spatial[0], edge_index_spatial[1]),
                  (edge_index_temporal[0], edge_index_temporal[1])]
    node_scores = jnp.zeros((n,), jnp.float32)
    for l in range(L):
        out = jnp.zeros_like(h)
        for t, (src, dst) in enumerate(edge_lists):
            o, asum = _gatv2_conv(h, src, dst, Wl[l, t], bl[l, t], Wr[l, t],
                                  br[l, t], att[l, t], bias_conv[l, t])
            out = out + o
            node_scores = node_scores + asum
        h = _layer_norm(h + out, ln_gamma[l], ln_beta[l])

    gate, gh, attn_scores = pl.pallas_call(
        _epilogue_kernel,
        out_shape=(
            jax.ShapeDtypeStruct((n, 1), jnp.float32),
            jax.ShapeDtypeStruct((n, D), jnp.float32),
            jax.ShapeDtypeStruct((n,), jnp.float32),
        ),
    )(h, W_gate, b_gate.reshape(1, 1), node_scores)
    gate = gate[:, 0]
    graph_emb = gh.sum(0) / (gate.sum() + 1e-8)
    return h, attn_scores, graph_emb


# speedup vs baseline: 7.0849x; 6.8408x over previous
import functools

import jax
import jax.numpy as jnp
from jax import lax
from jax.experimental import pallas as pl
from jax.experimental.pallas import tpu as pltpu
from jax.experimental.pallas import tpu_sc as plsc

N = 10000
E = 160000
D_IN = 128
D = 256
H = 8
C = D // H
L = 2

NC = 2
NS = 16
NW = NC * NS
CH = 40
CHP = 48
EA = E // NW
EB = E // NS
NCH_A = EA // CH
NCH_B = EB // CH
DSTRIDE = 624
DCHUNK = 64
NDUMP = 10

_MESH = plsc.VectorSubcoreMesh(core_axis_name="c", subcore_axis_name="s",
                               num_cores=NC, num_subcores=NS)
_SC_PARAMS = pltpu.CompilerParams(needs_layout_passes=False)


def _sanitize_idx(idx_ref, iota, n_valid=CH):
    for j in range(CHP // 16):
        off = 16 * j
        v = idx_ref[pl.ds(off, 16)]
        if (j + 1) * 16 > n_valid:
            v = jnp.where(iota < (n_valid - off), v, 0)
        idx_ref[pl.ds(off, 16)] = v


def _zero_rows(ref, nrows, zv):
    def zrow(i, _):
        for k in range(ref.shape[1] // 16):
            ref[i, pl.ds(16 * k, 16)] = zv
        return 0
    lax.fori_loop(0, nrows, zrow, 0)


def _passA_body(xl2_s, xr_s, att_s, src_s, dst_s,
                xl2_t, xr_t, att_t, src_t, dst_t,
                ex_s, ex_t, den2,
                idxa, idxb, idxd, xlo, xhi, xrb, exw, exsc, attv, zb,
                den_sh, sem):
    cid = lax.axis_index("c")
    sid = lax.axis_index("s")
    wid = sid * NC + cid
    iota = lax.iota(jnp.int32, 16)
    zv = jnp.zeros((16,), jnp.float32)

    _zero_rows(zb, DCHUNK, zv)
    for k in range(NDUMP):
        pltpu.sync_copy(zb, den_sh.at[pl.ds(sid * DSTRIDE + k * DCHUNK,
                                            DCHUNK)])
    _zero_rows(exsc, CHP, zv)
    plsc.subcore_barrier()

    def run_conv(xl2, xr, att_h, src, dst, ex_out, coff):
        pltpu.sync_copy(att_h, attv)
        att_rows = [attv[r] for r in range(16)]
        base0 = wid * EA

        def chunk(c, _):
            base = base0 + c * CH
            pltpu.sync_copy(src.at[pl.ds(base, CH)], idxa.at[pl.ds(0, CH)])
            pltpu.sync_copy(dst.at[pl.ds(base, CH)], idxd.at[pl.ds(0, CH)])
            _sanitize_idx(idxa, iota)
            _sanitize_idx(idxd, iota)
            for j in range(CHP // 16):
                off = 16 * j
                idxb[pl.ds(off, 16)] = idxa[pl.ds(off, 16)] + N
            cp1 = pltpu.async_copy(xl2.at[idxa], xlo, sem)
            cp2 = pltpu.async_copy(xl2.at[idxb], xhi, sem)
            cp3 = pltpu.async_copy(xr.at[idxd], xrb, sem)
            cp1.wait()
            cp2.wait()
            cp3.wait()

            def edge(e, _):
                sv = jnp.zeros((16,), jnp.float32)
                for h in range(H):
                    hb = xlo if h < 4 else xhi
                    ho = (h % 4) * 32
                    ro = h * 32
                    s0 = hb[e, pl.ds(ho, 16)] + xrb[e, pl.ds(ro, 16)]
                    s1 = hb[e, pl.ds(ho + 16, 16)] + xrb[e, pl.ds(ro + 16, 16)]
                    p = (jnp.maximum(s0, 0.2 * s0) * att_rows[2 * h]
                         + jnp.maximum(s1, 0.2 * s1) * att_rows[2 * h + 1])
                    sv = jnp.where(iota == h, jnp.sum(p), sv)
                ev = jnp.exp(sv)
                ev = jnp.where(iota < H, ev, 0.0)
                exw[e] = ev
                exsc[e, pl.ds(coff, 16)] = ev
                return 0
            lax.fori_loop(0, CH, edge, 0)

            pltpu.sync_copy(exw.at[pl.ds(0, CH)], ex_out.at[pl.ds(base, CH)])
            pltpu.sync_copy(exsc, den_sh.at[idxd], add=True)
            return 0
        lax.fori_loop(0, NCH_A, chunk, 0)

    run_conv(xl2_s, xr_s, att_s, src_s, dst_s, ex_s, 0)
    _zero_rows(exsc, CH, zv)
    run_conv(xl2_t, xr_t, att_t, src_t, dst_t, ex_t, 16)

    plsc.subcore_barrier()
    for k in range(NDUMP):
        sl = sid * DSTRIDE + k * DCHUNK
        pltpu.sync_copy(den_sh.at[pl.ds(sl, DCHUNK)],
                        den2.at[pl.ds(cid * N + sl, DCHUNK)])


_passA = functools.partial(
    pl.kernel,
    _passA_body,
    out_type=[
        jax.ShapeDtypeStruct((E, 16), jnp.float32),
        jax.ShapeDtypeStruct((E, 16), jnp.float32),
        jax.ShapeDtypeStruct((2 * N, 128), jnp.float32),
    ],
    mesh=_MESH,
    compiler_params=_SC_PARAMS,
    scratch_types=[
        pltpu.VMEM((CHP,), jnp.int32),
        pltpu.VMEM((CHP,), jnp.int32),
        pltpu.VMEM((CHP,), jnp.int32),
        pltpu.VMEM((CHP, 128), jnp.float32),
        pltpu.VMEM((CHP, 128), jnp.float32),
        pltpu.VMEM((CHP, 256), jnp.float32),
        pltpu.VMEM((CHP, 16), jnp.float32),
        pltpu.VMEM((CHP, 128), jnp.float32),
        pltpu.VMEM((16, 16), jnp.float32),
        pltpu.VMEM((DCHUNK, 128), jnp.float32),
        pltpu.VMEM_SHARED((N, 128), jnp.float32),
        pltpu.SemaphoreType.DMA,
    ],
)()


def _passB_body(xl2_s, ex_s, src_s, dst_s,
                xl2_t, ex_t, src_t, dst_t,
                out2_s, out2_t,
                idxs, idxd, xlb, exb, stage, zb, out_sh, sem):
    cid = lax.axis_index("c")
    sid = lax.axis_index("s")
    iota = lax.iota(jnp.int32, 16)
    zv = jnp.zeros((16,), jnp.float32)

    _zero_rows(zb, DCHUNK, zv)
    _zero_rows(stage, CHP, zv)

    head_masks = [iota == (4 * cid + j) for j in range(4)]
    row_off = cid * N

    def zero_accum():
        for k in range(NDUMP):
            pltpu.sync_copy(zb, out_sh.at[pl.ds(sid * DSTRIDE + k * DCHUNK,
                                                DCHUNK)])

    def dump_accum(out2):
        for k in range(NDUMP):
            sl = sid * DSTRIDE + k * DCHUNK
            pltpu.sync_copy(out_sh.at[pl.ds(sl, DCHUNK)],
                            out2.at[pl.ds(row_off + sl, DCHUNK)])

    def run_conv(xl2, ex, src, dst):
        base0 = sid * EB

        def chunk(c, _):
            base = base0 + c * CH
            pltpu.sync_copy(src.at[pl.ds(base, CH)], idxs.at[pl.ds(0, CH)])
            pltpu.sync_copy(dst.at[pl.ds(base, CH)], idxd.at[pl.ds(0, CH)])
            _sanitize_idx(idxs, iota)
            _sanitize_idx(idxd, iota)
            for j in range(CHP // 16):
                off = 16 * j
                idxs[pl.ds(off, 16)] = idxs[pl.ds(off, 16)] + row_off
            cp1 = pltpu.async_copy(xl2.at[idxs], xlb, sem)
            pltpu.sync_copy(ex.at[pl.ds(base, CH)], exb.at[pl.ds(0, CH)])
            cp1.wait()

            def edge(e, _):
                al = exb[e]
                for j in range(4):
                    aj = jnp.sum(jnp.where(head_masks[j], al, 0.0))
                    o = 32 * j
                    stage[e, pl.ds(o, 16)] = xlb[e, pl.ds(o, 16)] * aj
                    stage[e, pl.ds(o + 16, 16)] = xlb[e, pl.ds(o + 16, 16)] * aj
                return 0
            lax.fori_loop(0, CH, edge, 0)

            pltpu.sync_copy(stage, out_sh.at[idxd], add=True)
            return 0
        lax.fori_loop(0, NCH_B, chunk, 0)

    zero_accum()
    plsc.subcore_barrier()
    run_conv(xl2_s, ex_s, src_s, dst_s)
    plsc.subcore_barrier()
    dump_accum(out2_s)
    zero_accum()
    plsc.subcore_barrier()
    run_conv(xl2_t, ex_t, src_t, dst_t)
    plsc.subcore_barrier()
    dump_accum(out2_t)


_passB = functools.partial(
    pl.kernel,
    _passB_body,
    out_type=[
        jax.ShapeDtypeStruct((2 * N, 128), jnp.float32),
        jax.ShapeDtypeStruct((2 * N, 128), jnp.float32),
    ],
    mesh=_MESH,
    compiler_params=_SC_PARAMS,
    scratch_types=[
        pltpu.VMEM((CHP,), jnp.int32),
        pltpu.VMEM((CHP,), jnp.int32),
        pltpu.VMEM((CHP, 128), jnp.float32),
        pltpu.VMEM((CHP, 16), jnp.float32),
        pltpu.VMEM((CHP, 128), jnp.float32),
        pltpu.VMEM((DCHUNK, 128), jnp.float32),
        pltpu.VMEM_SHARED((N, 128), jnp.float32),
        pltpu.SemaphoreType.DMA,
    ],
)()


_RB = 1000


def _matmul_body(x_ref, w_ref, b_ref, o_ref):
    o_ref[...] = (jnp.dot(x_ref[...], w_ref[...],
                          preferred_element_type=jnp.float32) + b_ref[...])


def _matmul(x, w, b_row):
    n, k = x.shape
    m = w.shape[1]
    return pl.pallas_call(
        _matmul_body,
        grid=(n // _RB,),
        in_specs=[
            pl.BlockSpec((_RB, k), lambda i: (i, 0)),
            pl.BlockSpec((k, m), lambda i: (0, 0)),
            pl.BlockSpec((1, m), lambda i: (0, 0)),
        ],
        out_specs=pl.BlockSpec((_RB, m), lambda i: (i, 0)),
        out_shape=jax.ShapeDtypeStruct((n, m), jnp.float32),
    )(x, w, b_row.reshape(1, m))


def _ln_body(h_ref, rs_ref, ss_ref, rt_ref, st_ref, b_ref, g_ref, be_ref,
             out_ref):
    v = (h_ref[...] + rs_ref[...] * ss_ref[...] + rt_ref[...] * st_ref[...]
         + b_ref[...])
    mu = v.mean(-1, keepdims=True)
    var = ((v - mu) ** 2).mean(-1, keepdims=True)
    out_ref[...] = g_ref[...] * (v - mu) / jnp.sqrt(var + 1e-5) + be_ref[...]


def _ln_residual(h, raw_s, scale_s, raw_t, scale_t, bias_row, g, b):
    row = pl.BlockSpec((_RB, D), lambda i: (i, 0))
    one = pl.BlockSpec((1, D), lambda i: (0, 0))
    return pl.pallas_call(
        _ln_body,
        grid=(N // _RB,),
        in_specs=[row, row, row, row, row, one, one, one],
        out_specs=row,
        out_shape=jax.ShapeDtypeStruct((N, D), jnp.float32),
    )(h, raw_s, scale_s, raw_t, scale_t, bias_row.reshape(1, D),
      g.reshape(1, D), b.reshape(1, D))


def _epi_body(h_ref, wg_ref, bg_ref, ns_ref, gs_ref, gh_ref, ns_out_ref):
    h = h_ref[...]
    gate = jax.nn.sigmoid(h @ wg_ref[...] + bg_ref[0, 0])
    gs_ref[...] = jnp.sum(gate).reshape(1, 1)
    gh_ref[...] = jnp.sum(h * gate, axis=0, keepdims=True)
    ns = ns_ref[...]
    mx = jnp.max(ns)
    ns_out_ref[...] = jnp.where(mx > 0, ns / mx, ns)


def _split_halves(xl):
    return xl.reshape(N, 2, 128).transpose(1, 0, 2).reshape(2 * N, 128)


def _merge_halves(x2):
    return jnp.concatenate([x2[:N], x2[N:]], axis=1)


def kernel(x, query_embedding, W_in, b_in, W_q, b_q, Wl, bl, Wr, br, att,
           bias_conv, ln_gamma, ln_beta, W_gate, b_gate,
           edge_index_spatial, edge_index_temporal):
    src_s, dst_s = edge_index_spatial[0], edge_index_spatial[1]
    src_t, dst_t = edge_index_temporal[0], edge_index_temporal[1]

    qv = query_embedding @ W_q + b_q + b_in
    h = _matmul(x, W_in, qv)

    node_scores = jnp.zeros((N,), jnp.float32)
    for l in range(L):
        Wcat = jnp.concatenate(
            [Wl[l, 0], Wr[l, 0], Wl[l, 1], Wr[l, 1]], axis=1)
        bcat = jnp.concatenate(
            [bl[l, 0], br[l, 0], bl[l, 1], br[l, 1]], axis=0)
        proj = _matmul(h, Wcat, bcat)
        xl2_s = _split_halves(proj[:, 0:256])
        xr_s = proj[:, 256:512]
        xl2_t = _split_halves(proj[:, 512:768])
        xr_t = proj[:, 768:1024]
        att_s = att[l, 0].reshape(16, 16)
        att_t = att[l, 1].reshape(16, 16)

        ex_s, ex_t, den2 = _passA(
            xl2_s, xr_s, att_s, src_s, dst_s,
            xl2_t, xr_t, att_t, src_t, dst_t)

        den = den2[:N] + den2[N:]
        den_s = den[:, 0:H]
        den_t = den[:, 16:16 + H]
        node_scores = node_scores + (
            (den_s / (den_s + 1e-16)).sum(1) + (den_t / (den_t + 1e-16)).sum(1)
        ) / H
        scale_s = jnp.repeat(1.0 / (den_s + 1e-16), C, axis=1)
        scale_t = jnp.repeat(1.0 / (den_t + 1e-16), C, axis=1)

        out2_s, out2_t = _passB(
            xl2_s, ex_s, src_s, dst_s,
            xl2_t, ex_t, src_t, dst_t)
        h = _ln_residual(h, _merge_halves(out2_s), scale_s,
                         _merge_halves(out2_t), scale_t,
                         bias_conv[l, 0] + bias_conv[l, 1],
                         ln_gamma[l], ln_beta[l])

    gs, gh, attn_scores = pl.pallas_call(
        _epi_body,
        out_shape=(
            jax.ShapeDtypeStruct((1, 1), jnp.float32),
            jax.ShapeDtypeStruct((1, D), jnp.float32),
            jax.ShapeDtypeStruct((N,), jnp.float32),
        ),
    )(h, W_gate, b_gate.reshape(1, 1), node_scores)
    graph_emb = gh[0] / (gs[0, 0] + 1e-8)
    return h, attn_scores, graph_emb
